# Initial kernel scaffold; baseline (speedup 1.0000x reference)
import jax
import jax.numpy as jnp
from jax.experimental import pallas as pl


def kernel(x, edge_index, edge_attr, params):
    raise NotImplementedError("write your pallas kernel here")


# speedup vs baseline: 7.8402x; 7.8402x over previous
import functools

import jax
import jax.numpy as jnp
from jax import lax
from jax.experimental import pallas as pl
from jax.experimental.pallas import tpu as pltpu
from jax.experimental.pallas import tpu_sc as plsc

N = 10000
E = 320000
D = 128
H = 8
DH = 16
NPAD = 10112
ROW = 144
CHUNK = 16
NW = 32
NCH = 626
PER_W = NCH * CHUNK
EP = NW * PER_W
RPT = NPAD // 16

_mesh = plsc.VectorSubcoreMesh(core_axis_name="c", subcore_axis_name="s")


@functools.partial(
    pl.kernel,
    out_type=jax.ShapeDtypeStruct((2, NPAD, ROW), jnp.float32),
    mesh=_mesh,
    compiler_params=pltpu.CompilerParams(use_tc_tiling_on_sc=False,
                                         needs_layout_passes=False),
    scratch_types=[
        pltpu.VMEM((4, CHUNK), jnp.int32),
        pltpu.VMEM((4, CHUNK), jnp.int32),
        pltpu.VMEM((2, CHUNK, D), jnp.float32),
        pltpu.VMEM((2, CHUNK, 2 * D), jnp.float32),
        pltpu.VMEM((2, CHUNK, 2 * D), jnp.float32),
        pltpu.VMEM((2, CHUNK, ROW), jnp.float32),
        pltpu.VMEM_SHARED((NPAD, ROW), jnp.float32),
        pltpu.SemaphoreType.DMA,
        pltpu.SemaphoreType.DMA,
        pltpu.SemaphoreType.DMA,
        pltpu.SemaphoreType.DMA,
        pltpu.SemaphoreType.DMA,
        pltpu.SemaphoreType.DMA,
    ],
)
def _sc_edge(q_hbm, kv_hbm, keve_hbm, dst_hbm, src_hbm, out_hbm,
             dst_v, src_v, qb, kvb, evb, ob, acc,
             ix_sem0, ix_sem1, in_sem0, in_sem1, out_sem0, out_sem1):
    cid = lax.axis_index("c")
    sid = lax.axis_index("s")
    wid = sid * 2 + cid
    ix_sems = (ix_sem0, ix_sem1)
    in_sems = (in_sem0, in_sem1)
    out_sems = (out_sem0, out_sem1)
    ev16 = lax.iota(jnp.int32, 16)
    z16 = jnp.zeros((16,), jnp.float32)

    def zcol(cc, carry):
        col = jnp.full((16,), 0, jnp.int32) + cc
        plsc.store_scatter(ob.at[0], [ev16, col], z16)
        return carry

    lax.fori_loop(0, ROW, zcol, 0)
    for cc in range(D + H, ROW):
        plsc.store_scatter(ob.at[1], [ev16, jnp.full((16,), cc, jnp.int32)], z16)

    row0 = sid * RPT
    nfull = RPT // CHUNK
    for i in range(nfull):
        pltpu.sync_copy(ob.at[0], acc.at[pl.ds(row0 + i * CHUNK, CHUNK)])
    rem = RPT - nfull * CHUNK
    if rem:
        pltpu.sync_copy(ob.at[0].at[pl.ds(0, rem)],
                        acc.at[pl.ds(row0 + nfull * CHUNK, rem)])
    plsc.subcore_barrier()

    def start_idx(c, par):
        s = c % 4
        sem = ix_sems[par]
        pltpu.async_copy(dst_hbm.at[wid, c], dst_v.at[s], sem)
        pltpu.async_copy(src_hbm.at[wid, c], src_v.at[s], sem)

    def wait_idx(c, par):
        s = c % 4
        sem = ix_sems[par]
        pltpu.make_async_copy(dst_hbm.at[wid, c], dst_v.at[s], sem).wait()
        pltpu.make_async_copy(src_hbm.at[wid, c], src_v.at[s], sem).wait()

    def start_in(c, b):
        s = c % 4
        sem = in_sems[b]
        pltpu.async_copy(q_hbm.at[dst_v.at[s]], qb.at[b], sem)
        pltpu.async_copy(kv_hbm.at[src_v.at[s]], kvb.at[b], sem)
        pltpu.async_copy(keve_hbm.at[wid, c], evb.at[b], sem)

    def wait_in(c, b):
        s = c % 4
        sem = in_sems[b]
        pltpu.make_async_copy(q_hbm.at[dst_v.at[s]], qb.at[b], sem).wait()
        pltpu.make_async_copy(kv_hbm.at[src_v.at[s]], kvb.at[b], sem).wait()
        pltpu.make_async_copy(keve_hbm.at[wid, c], evb.at[b], sem).wait()

    def start_out(c, b):
        pltpu.async_copy(ob.at[b], acc.at[dst_v.at[c % 4]], out_sems[b],
                         add=True)

    def wait_out(c, b):
        pltpu.make_async_copy(ob.at[b], acc.at[dst_v.at[c % 4]],
                              out_sems[b]).wait()

    def compute(b):
        qq, kv, ee, oo = qb.at[b], kvb.at[b], evb.at[b], ob.at[b]
        for h in range(H):
            a = z16
            for j in range(DH):
                dcol = jnp.full((16,), h * DH + j, jnp.int32)
                qd = plsc.load_gather(qq, [ev16, dcol])
                kd = (plsc.load_gather(kv, [ev16, dcol])
                      + plsc.load_gather(ee, [ev16, dcol]))
                a = a + qd * kd
            exv = jnp.exp(a * 0.25)
            plsc.store_scatter(oo, [ev16, jnp.full((16,), D + h, jnp.int32)],
                               exv)
            for j in range(DH):
                d = h * DH + j
                vcol = jnp.full((16,), D + d, jnp.int32)
                vd = (plsc.load_gather(kv, [ev16, vcol])
                      + plsc.load_gather(ee, [ev16, vcol]))
                plsc.store_scatter(oo, [ev16, jnp.full((16,), d, jnp.int32)],
                                   vd * exv)

    start_idx(0, 0)
    start_idx(1, 1)
    wait_idx(0, 0)
    start_in(0, 0)

    def step(i, carry):
        for b in range(2):
            c = i * 2 + b

            @pl.when(c >= 2)
            def _():
                wait_out(c - 2, b)

            @pl.when(c + 2 < NCH)
            def _():
                start_idx(c + 2, b)

            wait_in(c, b)

            @pl.when(c + 1 < NCH)
            def _():
                wait_idx(c + 1, (b + 1) % 2)
                start_in(c + 1, (b + 1) % 2)

            compute(b)
            start_out(c, b)
        return carry

    lax.fori_loop(0, NCH // 2, step, 0)
    wait_out(NCH - 2, 0)
    wait_out(NCH - 1, 1)
    plsc.subcore_barrier()
    pltpu.sync_copy(acc.at[pl.ds(row0, RPT)],
                    out_hbm.at[cid].at[pl.ds(row0, RPT)])


def _ln(x, g, b):
    m = x.mean(-1, keepdims=True)
    v = ((x - m) ** 2).mean(-1, keepdims=True)
    return (x - m) / jnp.sqrt(v + 1e-5) * g + b


def kernel(x, edge_index, edge_attr, params):
    src = edge_index[0]
    dst = edge_index[1]
    e = params["emb"]

    pad = EP - E
    dst_p = jnp.concatenate([dst, jnp.full((pad,), N, jnp.int32)]).reshape(NW, NCH, CHUNK)
    src_p = jnp.concatenate([src, jnp.zeros((pad,), jnp.int32)]).reshape(NW, NCH, CHUNK)
    ea = jax.nn.relu(edge_attr @ e["W1"] + e["b1"]) @ e["W2"] + e["b2"]

    for p in params["layers"]:
        xn = _ln(x, p["g1"], p["n1"])
        Q = xn @ p["Wq"] + p["bq"]
        K = xn @ p["Wk"] + p["bk"]
        V = xn @ p["Wv"] + p["bv"]
        Ke = ea @ p["Wke"] + p["bke"]
        Ve = ea @ p["Wve"] + p["bve"]
        q_pad = jnp.pad(Q, ((0, NPAD - N), (0, 0)))
        kv_pad = jnp.pad(jnp.concatenate([K, V], axis=1), ((0, NPAD - N), (0, 0)))
        keve = jnp.pad(jnp.concatenate([Ke, Ve], axis=1), ((0, EP - E), (0, 0)))
        keve = keve.reshape(NW, NCH, CHUNK, 2 * D)

        parts = _sc_edge(q_pad, kv_pad, keve, dst_p, src_p)
        tot = (parts[0] + parts[1])[:N]
        aggU = tot[:, :D].reshape(N, H, DH)
        denom = tot[:, D:D + H]
        agg = (aggU / (denom[..., None] + 1e-16)).reshape(N, D)

        gate = jax.nn.sigmoid(agg @ p["Wih"] + p["bih"] + xn @ p["Whh"] + p["bhh"])
        upd = agg + gate * (xn @ p["Ws"] + p["bs"] - agg)
        x = x + (upd @ p["Wout"] + p["bout"])
        xn2 = _ln(x, p["g2"], p["n2"])
        h = jax.nn.relu(xn2 @ p["Wm1"] + p["bm1"]) @ p["Wm2"] + p["bm2"]
        x = x + h
    return x
